# baseline (speedup 1.0000x reference)
import jax
import jax.numpy as jnp
from jax.experimental import pallas as pl
from jax.experimental.pallas import tpu as pltpu

_DN = (((1,), (1,)), ((), ()))
_NCHUNK = 13


def _body(x_hbm, adj_hbm, w1_hbm, b1_ref, w2_hbm, b2_ref, fc1w_hbm,
          fc1b_ref, fc2w_ref, fc2b_ref, fc3w_ref, fc3b_ref, out_ref,
          wbuf, flat_s, xb, adjb, w1b, w2b, sems, sem_in):
    nout, kdim = wbuf.shape
    chunk = kdim // _NCHUNK
    cps = [pltpu.make_async_copy(src, dst, sem_in.at[i]) for i, (src, dst) in
           enumerate([(x_hbm, xb), (w1_hbm, w1b), (adj_hbm, adjb),
                      (w2_hbm, w2b)])]
    for cp in cps:
        cp.start()
    for k in range(_NCHUNK):
        sl = pl.ds(k * chunk, chunk)
        pltpu.make_async_copy(fc1w_hbm.at[:, sl], wbuf.at[:, sl],
                              sems.at[k]).start()

    cps[0].wait()
    cps[1].wait()
    s1 = jnp.dot(xb[...], w1b[...], preferred_element_type=jnp.float32)
    cps[2].wait()
    h1 = jax.nn.relu(
        jnp.dot(adjb[...], s1, preferred_element_type=jnp.float32)
        + b1_ref[...].reshape(1, -1)
    )
    cps[3].wait()
    s2 = jnp.dot(h1, w2b[...], preferred_element_type=jnp.float32)
    h2 = jax.nn.relu(
        jnp.dot(adjb[...], s2, preferred_element_type=jnp.float32)
        + b2_ref[...].reshape(1, -1)
    )
    n, nclass = h2.shape
    for r in range(n):
        flat_s[0:1, r * nclass:(r + 1) * nclass] = h2[r:r + 1, :]

    acc = jnp.zeros((nout, 128), jnp.float32)
    for k in range(_NCHUNK):
        sl = pl.ds(k * chunk, chunk)
        pltpu.make_async_copy(fc1w_hbm.at[:, sl], wbuf.at[:, sl],
                              sems.at[k]).wait()
        t = wbuf[:, sl] * flat_s[0:1, k * chunk:(k + 1) * chunk]
        for g in range(chunk // 128):
            acc = acc + t[:, g * 128:(g + 1) * 128]

    a1 = jax.nn.relu(acc.sum(axis=1).reshape(1, nout)
                     + fc1b_ref[...].reshape(1, -1))
    a2 = jax.nn.relu(
        jax.lax.dot_general(a1, fc2w_ref[...], _DN,
                            preferred_element_type=jnp.float32)
        + fc2b_ref[...].reshape(1, -1)
    )
    a3 = (jnp.sum(a2 * fc3w_ref[...], axis=1, keepdims=True)
          + fc3b_ref[...].reshape(1, -1))
    out_ref[...] = jax.nn.sigmoid(a3).reshape(1)


def kernel(x, adj, W1, b1, W2, b2, fc1_w, fc1_b, fc2_w, fc2_b, fc3_w, fc3_b):
    nout, kdim = fc1_w.shape
    vmem = pl.BlockSpec(memory_space=pltpu.MemorySpace.VMEM)

    y = pl.pallas_call(
        _body,
        in_specs=[pl.BlockSpec(memory_space=pl.ANY),
                  pl.BlockSpec(memory_space=pl.ANY),
                  pl.BlockSpec(memory_space=pl.ANY), vmem,
                  pl.BlockSpec(memory_space=pl.ANY), vmem,
                  pl.BlockSpec(memory_space=pl.ANY),
                  vmem, vmem, vmem, vmem, vmem],
        out_shape=jax.ShapeDtypeStruct((1,), jnp.float32),
        scratch_shapes=[
            pltpu.VMEM((nout, kdim), jnp.float32),
            pltpu.VMEM((1, kdim), jnp.float32),
            pltpu.VMEM(x.shape, jnp.float32),
            pltpu.VMEM(adj.shape, jnp.float32),
            pltpu.VMEM(W1.shape, jnp.float32),
            pltpu.VMEM(W2.shape, jnp.float32),
            pltpu.SemaphoreType.DMA((_NCHUNK,)),
            pltpu.SemaphoreType.DMA((4,)),
        ],
    )(x, adj, W1, b1, W2, b2, fc1_w, fc1_b, fc2_w, fc2_b, fc3_w, fc3_b)

    return y


# speedup vs baseline: 1.1203x; 1.1203x over previous
import jax
import jax.numpy as jnp
from jax.experimental import pallas as pl
from jax.experimental.pallas import tpu as pltpu

_DN = (((1,), (1,)), ((), ()))
_NCHUNK = 13


def _body(x_ref, adj_ref, w1_ref, b1_ref, w2_ref, b2_ref, fc1w_hbm,
          fc1b_ref, fc2w_ref, fc2b_ref, fc3w_ref, fc3b_ref, out_ref,
          wbuf_a, wbuf_b, flat_s, sems):
    nout = wbuf_a.shape[0]
    half = wbuf_a.shape[1]
    kdim = wbuf_a.shape[1] + wbuf_b.shape[1]
    chunk = kdim // _NCHUNK
    hchunk = half // chunk
    for k in range(_NCHUNK):
        buf, off = (wbuf_a, 0) if k < hchunk else (wbuf_b, half)
        sl_src = pl.ds(k * chunk, chunk)
        sl_dst = pl.ds(k * chunk - off, chunk)
        pltpu.make_async_copy(fc1w_hbm.at[:, sl_src], buf.at[:, sl_dst],
                              sems.at[k]).start()

    s1 = jnp.dot(x_ref[...], w1_ref[...], preferred_element_type=jnp.float32)
    h1 = jax.nn.relu(
        jnp.dot(adj_ref[...], s1, preferred_element_type=jnp.float32)
        + b1_ref[...].reshape(1, -1)
    )
    s2 = jnp.dot(h1, w2_ref[...], preferred_element_type=jnp.float32)
    h2 = jax.nn.relu(
        jnp.dot(adj_ref[...], s2, preferred_element_type=jnp.float32)
        + b2_ref[...].reshape(1, -1)
    )
    n, nclass = h2.shape
    for r in range(n):
        flat_s[0:1, r * nclass:(r + 1) * nclass] = h2[r:r + 1, :]

    acc = jnp.zeros((nout, 128), jnp.float32)
    for k in range(_NCHUNK):
        buf, off = (wbuf_a, 0) if k < hchunk else (wbuf_b, half)
        sl_src = pl.ds(k * chunk, chunk)
        sl_dst = pl.ds(k * chunk - off, chunk)
        pltpu.make_async_copy(fc1w_hbm.at[:, sl_src], buf.at[:, sl_dst],
                              sems.at[k]).wait()
        t = (buf[:, k * chunk - off:(k + 1) * chunk - off]
             * flat_s[0:1, k * chunk:(k + 1) * chunk])
        for g in range(chunk // 128):
            acc = acc + t[:, g * 128:(g + 1) * 128]

    a1 = jax.nn.relu(acc.sum(axis=1).reshape(1, nout)
                     + fc1b_ref[...].reshape(1, -1))
    a2 = jax.nn.relu(
        jax.lax.dot_general(a1, fc2w_ref[...], _DN,
                            preferred_element_type=jnp.float32)
        + fc2b_ref[...].reshape(1, -1)
    )
    a3 = (jnp.sum(a2 * fc3w_ref[...], axis=1, keepdims=True)
          + fc3b_ref[...].reshape(1, -1))
    out_ref[...] = jax.nn.sigmoid(a3).reshape(1)


def kernel(x, adj, W1, b1, W2, b2, fc1_w, fc1_b, fc2_w, fc2_b, fc3_w, fc3_b):
    nout, kdim = fc1_w.shape
    vmem = pl.BlockSpec(memory_space=pltpu.MemorySpace.VMEM)

    y = pl.pallas_call(
        _body,
        in_specs=[vmem, vmem, vmem, vmem, vmem, vmem,
                  pl.BlockSpec(memory_space=pl.ANY),
                  vmem, vmem, vmem, vmem, vmem],
        out_shape=jax.ShapeDtypeStruct((1,), jnp.float32),
        scratch_shapes=[
            pltpu.VMEM((nout, (kdim // _NCHUNK) * (_NCHUNK // 2 + 1)), jnp.float32),
            pltpu.VMEM((nout, (kdim // _NCHUNK) * (_NCHUNK - _NCHUNK // 2 - 1)), jnp.float32),
            pltpu.VMEM((1, kdim), jnp.float32),
            pltpu.SemaphoreType.DMA((_NCHUNK,)),
        ],
    )(x, adj, W1, b1, W2, b2, fc1_w, fc1_b, fc2_w, fc2_b, fc3_w, fc3_b)

    return y


# speedup vs baseline: 1.1266x; 1.0057x over previous
import jax
import jax.numpy as jnp
from jax.experimental import pallas as pl
from jax.experimental.pallas import tpu as pltpu

_DN = (((1,), (1,)), ((), ()))
_NCHUNK = 8


def _body(x_ref, adj_ref, w1_ref, b1_ref, w2_ref, b2_ref, fc1w_hbm,
          fc1b_ref, fc2w_ref, fc2b_ref, fc3w_ref, fc3b_ref, out_ref,
          wbuf, flat_s, sems):
    nout, kdim = wbuf.shape
    rows = nout // _NCHUNK
    for k in range(_NCHUNK):
        sl = pl.ds(k * rows, rows)
        pltpu.make_async_copy(fc1w_hbm.at[sl, :], wbuf.at[sl, :],
                              sems.at[k]).start()

    s1 = jnp.dot(x_ref[...], w1_ref[...], preferred_element_type=jnp.float32)
    h1 = jax.nn.relu(
        jnp.dot(adj_ref[...], s1, preferred_element_type=jnp.float32)
        + b1_ref[...].reshape(1, -1)
    )
    s2 = jnp.dot(h1, w2_ref[...], preferred_element_type=jnp.float32)
    h2 = jax.nn.relu(
        jnp.dot(adj_ref[...], s2, preferred_element_type=jnp.float32)
        + b2_ref[...].reshape(1, -1)
    )
    n, nclass = h2.shape
    for r in range(n):
        flat_s[0:1, r * nclass:(r + 1) * nclass] = h2[r:r + 1, :]

    accs = []
    for k in range(_NCHUNK):
        sl = pl.ds(k * rows, rows)
        pltpu.make_async_copy(fc1w_hbm.at[sl, :], wbuf.at[sl, :],
                              sems.at[k]).wait()
        t = wbuf[k * rows:(k + 1) * rows, :] * flat_s[0:1, :]
        accr = jnp.zeros((rows, 128), jnp.float32)
        for g in range(kdim // 128):
            accr = accr + t[:, g * 128:(g + 1) * 128]
        accs.append(accr)
    acc = jnp.concatenate(accs, axis=0)

    a1 = jax.nn.relu(acc.sum(axis=1).reshape(1, nout)
                     + fc1b_ref[...].reshape(1, -1))
    a2 = jax.nn.relu(
        jax.lax.dot_general(a1, fc2w_ref[...], _DN,
                            preferred_element_type=jnp.float32)
        + fc2b_ref[...].reshape(1, -1)
    )
    a3 = (jnp.sum(a2 * fc3w_ref[...], axis=1, keepdims=True)
          + fc3b_ref[...].reshape(1, -1))
    out_ref[...] = jax.nn.sigmoid(a3).reshape(1)


def kernel(x, adj, W1, b1, W2, b2, fc1_w, fc1_b, fc2_w, fc2_b, fc3_w, fc3_b):
    nout, kdim = fc1_w.shape
    vmem = pl.BlockSpec(memory_space=pltpu.MemorySpace.VMEM)

    y = pl.pallas_call(
        _body,
        in_specs=[vmem, vmem, vmem, vmem, vmem, vmem,
                  pl.BlockSpec(memory_space=pl.ANY),
                  vmem, vmem, vmem, vmem, vmem],
        out_shape=jax.ShapeDtypeStruct((1,), jnp.float32),
        scratch_shapes=[
            pltpu.VMEM((nout, kdim), jnp.float32),
            pltpu.VMEM((1, kdim), jnp.float32),
            pltpu.SemaphoreType.DMA((_NCHUNK,)),
        ],
    )(x, adj, W1, b1, W2, b2, fc1_w, fc1_b, fc2_w, fc2_b, fc3_w, fc3_b)

    return y
